# Initial kernel scaffold; baseline (speedup 1.0000x reference)
import jax
import jax.numpy as jnp
from jax.experimental import pallas as pl


def kernel(x_in, edge_index, gender, handed, W_ih0, W_hh0, b_ih0, b_hh0, W_ih1, W_hh1, b_ih1, b_hh1, W_ih2, W_hh2, b_ih2, b_hh2, Wg1, bg1, Wg2, bg2, Wg3, bg3, Wg4, bg4, Wf1, bf1, Wf2, bf2, Wf3, bf3):
    raise NotImplementedError("write your pallas kernel here")


# speedup vs baseline: 2.1343x; 2.1343x over previous
import functools

import jax
import jax.numpy as jnp
from jax import lax
from jax.experimental import pallas as pl
from jax.experimental.pallas import tpu as pltpu
from jax.experimental.pallas import tpu_sc as plsc

N = 10000
E = 160000
BS = 16
H = 640
G4 = 4 * H
NP = 10240
EP = 163840
EPT = EP // 32
EPC = EP // 16
CH = 128
NCH = EPC // CH
NPT = NP // 16
KP = 8960
BM = 512
NPB = NP // BM
TB = 400


def _mm_bias_kernel(x_ref, w_ref, b_ref, o_ref, acc_ref, *, nk):
    k = pl.program_id(2)

    @pl.when(k == 0)
    def _():
        acc_ref[...] = jnp.zeros_like(acc_ref)

    acc_ref[...] += jnp.dot(x_ref[...], w_ref[...],
                            preferred_element_type=jnp.float32)

    @pl.when(k == nk - 1)
    def _():
        o_ref[...] = acc_ref[...] + b_ref[...]


def _matmul_bias(x, w, b, bm, bn, bk):
    m, kdim = x.shape
    _, n = w.shape
    nk = kdim // bk
    return pl.pallas_call(
        functools.partial(_mm_bias_kernel, nk=nk),
        grid=(m // bm, n // bn, nk),
        in_specs=[
            pl.BlockSpec((bm, bk), lambda i, j, k: (i, k)),
            pl.BlockSpec((bk, bn), lambda i, j, k: (k, j)),
            pl.BlockSpec((1, bn), lambda i, j, k: (0, j)),
        ],
        out_specs=pl.BlockSpec((bm, bn), lambda i, j, k: (i, j)),
        out_shape=jax.ShapeDtypeStruct((m, n), jnp.float32),
        scratch_shapes=[pltpu.VMEM((bm, bn), jnp.float32)],
    )(x, w, b)


def _lstm_kernel(gx_ref, whh_ref, hs_ref, h_ref, c_ref):
    i = pl.program_id(0)

    @pl.when(i == 0)
    def _():
        h_ref[...] = jnp.zeros_like(h_ref)
        c_ref[...] = jnp.zeros_like(c_ref)

    def step(t, carry):
        h, c = carry
        g = gx_ref[pl.ds(t, 1), :] + jnp.dot(
            h, whh_ref[...], preferred_element_type=jnp.float32)
        ig = jax.nn.sigmoid(g[:, 0:H])
        fg = jax.nn.sigmoid(g[:, H:2 * H])
        gg = jnp.tanh(g[:, 2 * H:3 * H])
        og = jax.nn.sigmoid(g[:, 3 * H:4 * H])
        c = fg * c + ig * gg
        h = og * jnp.tanh(c)
        hs_ref[pl.ds(t, 1), :] = h
        return (h, c)

    h, c = lax.fori_loop(0, TB, step, (h_ref[...], c_ref[...]))
    h_ref[...] = h
    c_ref[...] = c


def _lstm_scan(gx, whh_t):
    t = gx.shape[0]
    return pl.pallas_call(
        _lstm_kernel,
        grid=(t // TB,),
        in_specs=[
            pl.BlockSpec((TB, G4), lambda i: (i, 0)),
            pl.BlockSpec((H, G4), lambda i: (0, 0)),
        ],
        out_specs=pl.BlockSpec((TB, H), lambda i: (i, 0)),
        out_shape=jax.ShapeDtypeStruct((t, H), jnp.float32),
        scratch_shapes=[pltpu.VMEM((1, H), jnp.float32),
                        pltpu.VMEM((1, H), jnp.float32)],
    )(gx, whh_t)


_MESH = plsc.VectorSubcoreMesh(core_axis_name="c", subcore_axis_name="s")
DEGW = 16


def _deg_kernel(dst_hbm, ones_hbm, zeros_hbm, out_hbm, dst_v, ones_v, deg_sh):
    cid = lax.axis_index("c")
    sid = lax.axis_index("s")
    pltpu.sync_copy(dst_hbm.at[sid], dst_v)
    pltpu.sync_copy(ones_hbm, ones_v)
    pltpu.sync_copy(zeros_hbm.at[pl.ds(sid * NPT, NPT)],
                    deg_sh.at[pl.ds(sid * NPT, NPT)])
    plsc.subcore_barrier()
    half = NCH // 2

    def chunk(g, carry):
        pltpu.sync_copy(ones_v, deg_sh.at[dst_v.at[cid * half + g]], add=True)
        return carry

    lax.fori_loop(0, half, chunk, 0)
    plsc.subcore_barrier()
    pltpu.sync_copy(deg_sh.at[pl.ds(sid * NPT, NPT)],
                    out_hbm.at[pl.ds(cid * NP + sid * NPT, NPT)])


_SC_PARAMS = pltpu.CompilerParams(use_tc_tiling_on_sc=False)


def _deg_partials(dst3):
    k = pl.kernel(
        _deg_kernel,
        mesh=_MESH,
        out_type=jax.ShapeDtypeStruct((2 * NP, DEGW), jnp.float32),
        scratch_types=[pltpu.VMEM((NCH, CH), jnp.int32),
                       pltpu.VMEM((CH, DEGW), jnp.float32),
                       pltpu.VMEM_SHARED((NP, DEGW), jnp.float32)],
        compiler_params=_SC_PARAMS,
    )
    return k(dst3, jnp.ones((CH, DEGW), jnp.float32),
             jnp.zeros((NP, DEGW), jnp.float32))


def _dis_kernel(d_ref, o_ref, *, bm):
    i = pl.program_id(0)
    d = d_ref[...]
    s = (d[0] + d[1])[:, 0:1]
    rows = i * bm + lax.broadcasted_iota(jnp.int32, (bm, 1), 0)
    o_ref[...] = jnp.where(rows < N, lax.rsqrt(s + 1.0), 0.0)


def _dis_from_partials(deg2):
    return pl.pallas_call(
        functools.partial(_dis_kernel, bm=BM),
        grid=(NPB,),
        in_specs=[pl.BlockSpec((2, BM, DEGW), lambda i: (0, i, 0))],
        out_specs=pl.BlockSpec((BM, 1), lambda i: (i, 0)),
        out_shape=jax.ShapeDtypeStruct((NP, 1), jnp.float32),
    )(deg2)


def _xws_kernel(x_ref, w_ref, dis_ref, o_ref):
    o_ref[...] = dis_ref[...] * jnp.dot(
        x_ref[...], w_ref[...], preferred_element_type=jnp.float32)


def _xws_matmul(x, w, dis2d, dp):
    din = x.shape[1]
    return pl.pallas_call(
        _xws_kernel,
        grid=(NPB,),
        in_specs=[
            pl.BlockSpec((BM, din), lambda i: (i, 0)),
            pl.BlockSpec((din, dp), lambda i: (0, 0)),
            pl.BlockSpec((BM, 1), lambda i: (i, 0)),
        ],
        out_specs=pl.BlockSpec((BM, dp), lambda i: (i, 0)),
        out_shape=jax.ShapeDtypeStruct((NP, dp), jnp.float32),
    )(x, w, dis2d)


def _gcn_agg_kernel(src_hbm, dst_hbm, xws_hbm, zeros_hbm, out_hbm,
                    src_v, dst_v, rows_v, agg_sh, sem):
    cid = lax.axis_index("c")
    sid = lax.axis_index("s")
    pltpu.sync_copy(src_hbm.at[cid, sid], src_v)
    pltpu.sync_copy(dst_hbm.at[sid], dst_v)
    pltpu.sync_copy(zeros_hbm.at[pl.ds(sid * NPT, NPT)],
                    agg_sh.at[pl.ds(sid * NPT, NPT)])
    plsc.subcore_barrier()

    def chunk(g, carry):
        pltpu.async_copy(xws_hbm.at[src_v.at[g]], rows_v, sem).wait()
        pltpu.sync_copy(rows_v, agg_sh.at[dst_v.at[g]], add=True)
        return carry

    lax.fori_loop(0, NCH, chunk, 0)
    plsc.subcore_barrier()
    pltpu.sync_copy(agg_sh.at[pl.ds(sid * NPT, NPT)],
                    out_hbm.at[pl.ds(cid * NP + sid * NPT, NPT)])


def _gcn_agg(src3, dst3, xws, zeros, dh):
    k = pl.kernel(
        _gcn_agg_kernel,
        mesh=_MESH,
        out_type=jax.ShapeDtypeStruct((2 * NP, dh), jnp.float32),
        scratch_types=[pltpu.VMEM((NCH, CH), jnp.int32),
                       pltpu.VMEM((NCH, CH), jnp.int32),
                       pltpu.VMEM((CH, dh), jnp.float32),
                       pltpu.VMEM_SHARED((NP, dh), jnp.float32),
                       pltpu.SemaphoreType.DMA],
        compiler_params=_SC_PARAMS,
    )
    return k(src3, dst3, xws, zeros)


def _post_kernel(agg_ref, xws_ref, dis_ref, b_ref, o_ref, s1_ref, s2_ref,
                 *, bm, dp):
    p = pl.program_id(0)
    i = pl.program_id(1)

    agg = agg_ref[...]
    xws = xws_ref[...]
    npieces = agg.shape[0]
    y = jnp.concatenate([agg[k] + xws[k] for k in range(npieces)], axis=1)
    y = y * dis_ref[...] + b_ref[...]
    y = jnp.where(y >= 0.0, y, 0.01 * y)
    rows = i * bm + lax.broadcasted_iota(jnp.int32, (bm, 1), 0)
    y = jnp.where(rows < N, y, 0.0)

    @pl.when(p == 0)
    def _():
        @pl.when(i == 0)
        def _():
            s1_ref[...] = jnp.zeros((1, dp), jnp.float32)
            s2_ref[...] = jnp.zeros((1, dp), jnp.float32)

        s1_ref[...] += jnp.sum(y, axis=0, keepdims=True)
        s2_ref[...] += jnp.sum(y * y, axis=0, keepdims=True)

    @pl.when(p == 1)
    def _():
        m = s1_ref[...] * (1.0 / N)
        v = s2_ref[...] * (1.0 / N) - m * m
        o_ref[...] = (y - m) * lax.rsqrt(v + 1e-5)


def _post(agg2, xws2, dis2d, b1dp, dp):
    npieces, _, dq = agg2.shape
    return pl.pallas_call(
        functools.partial(_post_kernel, bm=BM, dp=dp),
        grid=(2, NPB),
        in_specs=[
            pl.BlockSpec((npieces, BM, dq), lambda p, i: (0, i, 0)),
            pl.BlockSpec((npieces, BM, dq), lambda p, i: (0, i, 0)),
            pl.BlockSpec((BM, 1), lambda p, i: (i, 0)),
            pl.BlockSpec((1, dp), lambda p, i: (0, 0)),
        ],
        out_specs=pl.BlockSpec((BM, dp), lambda p, i: (i, 0)),
        out_shape=jax.ShapeDtypeStruct((NP, dp), jnp.float32),
        scratch_shapes=[pltpu.VMEM((1, dp), jnp.float32),
                        pltpu.VMEM((1, dp), jnp.float32)],
    )(agg2, xws2, dis2d, b1dp)


def _head_kernel(x_ref, g_ref, hd_ref, w1_ref, b1_ref, w2_ref, b2_ref,
                 w3_ref, b3_ref, o_ref):
    blen = N // BS
    jj = lax.broadcasted_iota(jnp.int32, (BS, NP), 1)
    bb = lax.broadcasted_iota(jnp.int32, (BS, NP), 0)
    seg = jnp.where((jj // blen) == bb, 1.0, 0.0)
    pooled = jnp.dot(seg, x_ref[...], preferred_element_type=jnp.float32,
                     precision=lax.Precision.HIGHEST)
    col = lax.broadcasted_iota(jnp.int32, (BS, 64), 1)
    xcat = pooled + jnp.where(col == 50, g_ref[...], 0.0) \
        + jnp.where(col == 51, hd_ref[...], 0.0)
    a = jnp.dot(xcat, w1_ref[...], preferred_element_type=jnp.float32) \
        + b1_ref[...]
    a = jnp.dot(a, w2_ref[...], preferred_element_type=jnp.float32) \
        + b2_ref[...]
    o_ref[...] = jnp.dot(a, w3_ref[...],
                         preferred_element_type=jnp.float32) + b3_ref[...]


def _head(x4, gender, handed, w1, b1, w2, b2, w3, b3):
    return pl.pallas_call(
        _head_kernel,
        out_shape=jax.ShapeDtypeStruct((BS, 1), jnp.float32),
    )(x4, gender, handed, w1, b1, w2, b2, w3, b3)


def kernel(x_in, edge_index, gender, handed,
           W_ih0, W_hh0, b_ih0, b_hh0, W_ih1, W_hh1, b_ih1, b_hh1,
           W_ih2, W_hh2, b_ih2, b_hh2,
           Wg1, bg1, Wg2, bg2, Wg3, bg3, Wg4, bg4,
           Wf1, bf1, Wf2, bf2, Wf3, bf3):
    f32 = jnp.float32
    lenin = x_in.shape[1]

    xp = jnp.pad(x_in, ((0, 0), (0, KP - lenin)))
    w0t = jnp.pad(W_ih0, ((0, 0), (0, KP - lenin))).T
    gx = _matmul_bias(xp, w0t, (b_ih0 + b_hh0).reshape(1, G4),
                      bm=400, bn=640, bk=1280)
    hs = _lstm_scan(gx, W_hh0.T)
    gx = _matmul_bias(hs, W_ih1.T, (b_ih1 + b_hh1).reshape(1, G4),
                      bm=400, bn=1280, bk=640)
    hs = _lstm_scan(gx, W_hh1.T)
    gx = _matmul_bias(hs, W_ih2.T, (b_ih2 + b_hh2).reshape(1, G4),
                      bm=400, bn=1280, bk=640)
    hs = _lstm_scan(gx, W_hh2.T)

    src = edge_index[0].astype(jnp.int32)
    dst = edge_index[1].astype(jnp.int32)
    pad_rows = (jnp.arange(EP - E, dtype=jnp.int32) % (NP - N)) + N
    srcp = jnp.concatenate([src, pad_rows])
    dstp = jnp.concatenate([dst, pad_rows])
    src3 = jnp.stack([srcp, srcp + NP]).reshape(2, 16, NCH, CH)
    dst3 = dstp.reshape(16, NCH, CH)

    deg2 = _deg_partials(dst3).reshape(2, NP, DEGW)
    dis2d = _dis_from_partials(deg2)

    x = jnp.pad(hs, ((0, NP - N), (0, 0)))
    gdims = [(Wg1, bg1, 320, 320, 4), (Wg2, bg2, 180, 192, 2),
             (Wg3, bg3, 90, 96, 2), (Wg4, bg4, 50, 64, 2)]
    for wg, bg, dtrue, dp, npieces in gdims:
        dq = dp // npieces
        din = x.shape[1]
        wp = jnp.pad(wg, ((0, din - wg.shape[0]), (0, dp - dtrue)))
        bp = jnp.pad(bg, (0, dp - dtrue)).reshape(1, dp)
        xws = _xws_matmul(x, wp, dis2d, dp)
        xwsp = jnp.stack([xws[:, k * dq:(k + 1) * dq]
                          for k in range(npieces)])
        zeros = jnp.zeros((NP, dq), f32)
        aggs = [_gcn_agg(src3, dst3,
                         xwsp[2 * s:2 * s + 2].reshape(2 * NP, dq),
                         zeros, dq).reshape(2, NP, dq)
                for s in range(npieces // 2)]
        agg = jnp.concatenate(aggs, axis=0)
        x = _post(agg, xwsp, dis2d, bp, dp)

    w1 = jnp.pad(Wf1, ((0, 64 - Wf1.shape[0]), (0, 0)))
    return _head(x, gender, handed, w1, bf1.reshape(1, -1),
                 Wf2, bf2.reshape(1, -1), Wf3, bf3.reshape(1, -1))
